# baseline (speedup 1.0000x reference)
import jax
import jax.numpy as jnp
from jax import lax
from jax.experimental import pallas as pl
from jax.experimental.pallas import tpu as pltpu
from jax.experimental.pallas import tpu_sc as plsc

_BATCH = 4096
_N_FIELDS = 26
_PER_FIELD_VOCAB = 50
_EMBED_DIM = 64

_NC = 2
_NS = 16
_NW = _NC * _NS

_B_FLAT = _BATCH * _N_FIELDS
_PER_W = _B_FLAT // _NW
_SAMPLES_W = _BATCH // _NW
_MB = 4
_MROWS = _MB * _N_FIELDS
_N_MACRO = _SAMPLES_W // _MB
_NBUF = 4
_LAG = 2
_LANES = 16


def _body(cat_hbm, table_hbm, out_hbm, cat_v, idx_v, bufs, gsems, wsems):
    wid = lax.axis_index("s") * _NC + lax.axis_index("c")

    pltpu.sync_copy(cat_hbm.at[pl.ds(wid * _PER_W, _PER_W)], cat_v)

    lane = lax.broadcasted_iota(jnp.int32, (_LANES,), 0)
    for j in range(_PER_W // _LANES):
        p = j * _LANES
        off = ((p + lane) % _N_FIELDS) * _PER_FIELD_VOCAB
        idx_v[pl.ds(p, _LANES)] = cat_v[pl.ds(p, _LANES)] + off

    pend_g = [None] * _NBUF
    pend_w = [None] * _NBUF
    for m in range(_N_MACRO + _LAG):
        if m < _N_MACRO:
            s = m % _NBUF
            if pend_w[s] is not None:
                for c in pend_w[s]:
                    c.wait()
            pend_g[s] = pltpu.async_copy(
                table_hbm.at[idx_v.at[pl.ds(m * _MROWS, _MROWS)]],
                bufs[s],
                gsems[s],
            )
        i = m - _LAG
        if 0 <= i < _N_MACRO:
            s = i % _NBUF
            pend_g[s].wait()
            b0 = wid * _SAMPLES_W + i * _MB
            pend_w[s] = tuple(
                pltpu.async_copy(
                    bufs[s].at[pl.ds(q * _N_FIELDS, _N_FIELDS)],
                    out_hbm.at[b0 + q],
                    wsems[s],
                )
                for q in range(_MB)
            )
    for s in range(_NBUF):
        if pend_w[s] is not None:
            for c in pend_w[s]:
                c.wait()


@jax.jit
def _lookup(cat_flat, table):
    mesh = plsc.VectorSubcoreMesh(
        core_axis_name="c", subcore_axis_name="s", num_cores=_NC, num_subcores=_NS
    )
    k = pl.kernel(
        _body,
        out_type=jax.ShapeDtypeStruct((_BATCH, _N_FIELDS, _EMBED_DIM), jnp.float32),
        mesh=mesh,
        scratch_types=[
            pltpu.VMEM((_PER_W,), jnp.int32),
            pltpu.VMEM((_PER_W,), jnp.int32),
            [pltpu.VMEM((_MROWS, _EMBED_DIM), jnp.float32) for _ in range(_NBUF)],
            [pltpu.SemaphoreType.DMA for _ in range(_NBUF)],
            [pltpu.SemaphoreType.DMA for _ in range(_NBUF)],
        ],
        compiler_params=pltpu.CompilerParams(use_tc_tiling_on_sc=False),
    )
    return k(cat_flat, table)


def kernel(cat, table):
    return _lookup(cat.reshape(_B_FLAT), table)
